# baseline (speedup 1.0000x reference)
import functools

import jax
import jax.numpy as jnp
from jax import lax
from jax.experimental import pallas as pl
from jax.experimental.pallas import tpu as pltpu
from jax.experimental.pallas import tpu_sc as plsc

BATCH = 16384
LABEL_DIM = 32
TEXT_DIM = 768

NC = 2
NS = 16
NW = NC * NS
B_PER_W = BATCH // NW
GRP = 16
NGRP = B_PER_W // GRP

_MESH = plsc.VectorSubcoreMesh(core_axis_name="c", subcore_axis_name="s")


@functools.partial(
    pl.kernel,
    mesh=_MESH,
    out_type=jax.ShapeDtypeStruct((BATCH, LABEL_DIM), jnp.float32),
    scratch_types=[
        pltpu.VMEM((B_PER_W,), jnp.int32),
        pltpu.SemaphoreType.DMA,
    ],
    compiler_params=pltpu.CompilerParams(needs_layout_passes=False),
)
def _sc_gather(table_hbm, idx_hbm, out_hbm, idx_v, sem):
    wid = lax.axis_index("s") * NC + lax.axis_index("c")
    base = wid * B_PER_W
    pltpu.sync_copy(idx_hbm.at[pl.ds(base, B_PER_W)], idx_v)
    lanes = lax.iota(jnp.int32, GRP)

    def one_group(g, _):
        v = idx_v[pl.ds(g * GRP, GRP)]
        for l in range(GRP):
            row = lax.reduce_sum_p.bind(
                jnp.where(lanes == l, v, 0), axes=(0,))
            pltpu.make_async_copy(
                table_hbm.at[pl.ds(row, 1)],
                out_hbm.at[pl.ds(base + g * GRP + l, 1)],
                sem,
            ).start()
        return 0

    lax.fori_loop(0, NGRP, one_group, 0)
    pltpu.make_async_copy(
        table_hbm.at[pl.ds(0, B_PER_W)],
        out_hbm.at[pl.ds(base, B_PER_W)],
        sem,
    ).wait()


def _mm_body(emb_ref, w_ref, b_ref, out_ref):
    out_ref[...] = lax.dot_general(
        emb_ref[...], w_ref[...],
        (((1,), (1,)), ((), ())),
        preferred_element_type=jnp.float32,
    ) + b_ref[...]


BM = 1024


def kernel(labels, label_emb, W, b):
    emb = _sc_gather(label_emb, labels)
    b2d = b.reshape(1, TEXT_DIM)
    out = pl.pallas_call(
        _mm_body,
        grid=(BATCH // BM,),
        in_specs=[
            pl.BlockSpec((BM, LABEL_DIM), lambda i: (i, 0)),
            pl.BlockSpec((TEXT_DIM, LABEL_DIM), lambda i: (0, 0)),
            pl.BlockSpec((1, TEXT_DIM), lambda i: (0, 0)),
        ],
        out_specs=pl.BlockSpec((BM, TEXT_DIM), lambda i: (i, 0)),
        out_shape=jax.ShapeDtypeStruct((BATCH, TEXT_DIM), jnp.float32),
    )(emb, W, b2d)
    return out


# speedup vs baseline: 1.6122x; 1.6122x over previous
import functools

import jax
import jax.numpy as jnp
from jax import lax
from jax.experimental import pallas as pl
from jax.experimental.pallas import tpu as pltpu

BATCH = 16384
LABEL_DIM = 32
TEXT_DIM = 768

BM = 2048
NBLK = BATCH // BM
NBUF = 2


def _issue_rows(labels_smem, table_hbm, emb_v, sem, blk):
    base = blk * BM

    def issue(j, _):
        row = labels_smem[base + j]
        pltpu.make_async_copy(
            table_hbm.at[pl.ds(row, 1)], emb_v.at[pl.ds(j, 1)], sem
        ).start()
        return 0

    lax.fori_loop(0, BM, issue, 0, unroll=8)


def _body(labels_smem, table_hbm, w_ref, b_ref, out_ref, emb_v, sem):
    i = pl.program_id(0)

    @pl.when(i == 0)
    def _prologue():
        _issue_rows(labels_smem, table_hbm, emb_v.at[0], sem.at[0], 0)

    @pl.when(i + 1 < NBLK)
    def _next():
        _issue_rows(labels_smem, table_hbm, emb_v.at[(i + 1) % NBUF],
                    sem.at[(i + 1) % NBUF], i + 1)

    pltpu.make_async_copy(
        table_hbm.at[pl.ds(0, BM)], emb_v.at[i % NBUF], sem.at[i % NBUF]
    ).wait()
    out_ref[...] = lax.dot_general(
        emb_v[i % NBUF], w_ref[...],
        (((1,), (1,)), ((), ())),
        preferred_element_type=jnp.float32,
    ) + b_ref[...]


def kernel(labels, label_emb, W, b):
    b2d = b.reshape(1, TEXT_DIM)
    grid_spec = pltpu.PrefetchScalarGridSpec(
        num_scalar_prefetch=1,
        grid=(NBLK,),
        in_specs=[
            pl.BlockSpec(memory_space=pl.ANY),
            pl.BlockSpec((TEXT_DIM, LABEL_DIM), lambda i, *_: (0, 0)),
            pl.BlockSpec((1, TEXT_DIM), lambda i, *_: (0, 0)),
        ],
        out_specs=pl.BlockSpec((BM, TEXT_DIM), lambda i, *_: (i, 0)),
        scratch_shapes=[
            pltpu.VMEM((NBUF, BM, LABEL_DIM), jnp.float32),
            pltpu.SemaphoreType.DMA((NBUF,)),
        ],
    )
    out = pl.pallas_call(
        _body,
        grid_spec=grid_spec,
        out_shape=jax.ShapeDtypeStruct((BATCH, TEXT_DIM), jnp.float32),
    )(labels, label_emb, W, b2d)
    return out
